# baseline (speedup 1.0000x reference)
import functools

import jax
import jax.numpy as jnp
from jax import lax
from jax.experimental import pallas as pl
from jax.experimental.pallas import tpu as pltpu
from jax.experimental.pallas import tpu_sc as plsc

B, L = 16384, 50
V, D = 1000000, 64
H, C = 64, 2

NW = 32
SEQ_PER_W = B // NW
S_CHUNK = 8
CT = S_CHUNK * L
NCH = SEQ_PER_W // S_CHUNK
GATHER_SPLITS = [(0, 128), (128, 128), (256, 128), (384, 16)]
LANES = 16


def _sc_pool_body(ids_hbm, mask_hbm, table_hbm, out_hbm,
                  idx0, idx1, m0, m1, rows0, rows1, wbuf, outv, sem0, sem1):
    c = lax.axis_index("c")
    s = lax.axis_index("s")
    wid = s * 2 + c
    tok_base = wid * (SEQ_PER_W * L)
    seq_base = wid * SEQ_PER_W

    idx = [idx0, idx1]
    msk = [m0, m1]
    rows = [rows0, rows1]
    sems = [sem0, sem1]

    def load_and_fire(k, b):
        tb = tok_base + k * CT
        pltpu.sync_copy(ids_hbm.at[pl.ds(tb, CT)], idx[b])
        pltpu.sync_copy(mask_hbm.at[pl.ds(tb, CT)], msk[b])
        for (o, n) in GATHER_SPLITS:
            pltpu.async_copy(table_hbm.at[idx[b].at[pl.ds(o, n)]],
                             rows[b].at[pl.ds(o, n)], sems[b])

    def wait_gathers(b):
        for (o, n) in GATHER_SPLITS:
            pltpu.make_async_copy(table_hbm.at[idx[b].at[pl.ds(o, n)]],
                                  rows[b].at[pl.ds(o, n)], sems[b]).wait()

    def compute_weights(b):
        for t in range(CT // LANES):
            ids16 = idx[b][pl.ds(t * LANES, LANES)]
            mm16 = msk[b][pl.ds(t * LANES, LANES)]
            keep = (ids16 != 0) & (mm16 != 0)
            wbuf[pl.ds(t * LANES, LANES)] = jnp.where(keep, 1.0, 0.0)

    def accumulate(k, b):
        def seq_body(si, carry):
            tb = si * L
            wch = [wbuf[pl.ds(tb, LANES)],
                   wbuf[pl.ds(tb + 16, LANES)],
                   wbuf[pl.ds(tb + 32, LANES)],
                   wbuf[pl.ds(tb + 34, LANES)]]
            acc = [jnp.zeros((LANES,), jnp.float32) for _ in range(4)]
            for l in range(L):
                w = wch[l // 16][l % 16] if l < 48 else wch[3][l - 34]
                for j in range(4):
                    acc[j] = acc[j] + w * rows[b][tb + l, pl.ds(j * LANES, LANES)]
            for j in range(4):
                outv[si, pl.ds(j * LANES, LANES)] = acc[j]
            return carry
        lax.fori_loop(0, S_CHUNK, seq_body, 0)
        pltpu.sync_copy(outv, out_hbm.at[pl.ds(seq_base + k * S_CHUNK, S_CHUNK)])

    load_and_fire(0, 0)

    def pair_body(kk, carry):
        for b in (0, 1):
            k = kk * 2 + b

            @pl.when(k + 1 < NCH)
            def _():
                load_and_fire(k + 1, 1 - b)

            compute_weights(b)
            wait_gathers(b)
            accumulate(k, b)
        return carry

    lax.fori_loop(0, NCH // 2, pair_body, 0)


@functools.partial(jax.jit, static_argnames=())
def _sc_pool(ids_flat, mask_flat, table):
    kfn = pl.kernel(
        _sc_pool_body,
        out_type=jax.ShapeDtypeStruct((B, D), jnp.float32),
        mesh=plsc.VectorSubcoreMesh(core_axis_name="c", subcore_axis_name="s"),
        compiler_params=pltpu.CompilerParams(use_tc_tiling_on_sc=False),
        scratch_types=[
            pltpu.VMEM((CT,), jnp.int32),
            pltpu.VMEM((CT,), jnp.int32),
            pltpu.VMEM((CT,), jnp.int32),
            pltpu.VMEM((CT,), jnp.int32),
            pltpu.VMEM((CT, D), jnp.float32),
            pltpu.VMEM((CT, D), jnp.float32),
            pltpu.VMEM((CT,), jnp.float32),
            pltpu.VMEM((S_CHUNK, D), jnp.float32),
            pltpu.SemaphoreType.DMA,
            pltpu.SemaphoreType.DMA,
        ],
    )
    return kfn(ids_flat, mask_flat, table)


TBLK = 128
NFULL = V // TBLK
VREM = V - NFULL * TBLK
ITERS_PER_W = NFULL // NW
EXTRA_BASE = ITERS_PER_W * NW


def _sc_detile_body(tableT_hbm, tail_hbm, out_hbm, blk0, blk1, ob0, ob1,
                    isem0, isem1, osem0, osem1):
    c = lax.axis_index("c")
    s = lax.axis_index("s")
    wid = s * 2 + c

    blks = [blk0, blk1]
    obs = [ob0, ob1]
    isems = [isem0, isem1]
    osems = [osem0, osem1]

    row16 = [lax.iota(jnp.int32, LANES) + cc * LANES for cc in range(4)]

    def fire_in(k, b):
        j = wid + NW * k
        off = pl.multiple_of(j * TBLK, TBLK)
        pltpu.async_copy(tableT_hbm.at[pl.ds(0, D), pl.ds(off, TBLK)],
                         blks[b], isems[b])

    def wait_in(k, b):
        j = wid + NW * k
        off = pl.multiple_of(j * TBLK, TBLK)
        pltpu.make_async_copy(
            tableT_hbm.at[pl.ds(0, D), pl.ds(off, TBLK)],
            blks[b], isems[b]).wait()

    def fire_out(k, b):
        j = wid + NW * k
        pltpu.async_copy(obs[b], out_hbm.at[pl.ds(j * D, D)], osems[b])

    def wait_out(k, b):
        j = wid + NW * k
        pltpu.make_async_copy(
            obs[b], out_hbm.at[pl.ds(j * D, D)], osems[b]).wait()

    def transpose(b):
        blk, ob = blks[b], obs[b]

        def row_body(r, carry):
            v0 = jnp.full((LANES,), 2 * r, jnp.int32)
            v1 = jnp.full((LANES,), 2 * r + 1, jnp.int32)
            for cc in range(4):
                ob[r, pl.ds(cc * LANES, LANES)] = plsc.load_gather(
                    blk, [row16[cc], v0])
                ob[r, pl.ds(D + cc * LANES, LANES)] = plsc.load_gather(
                    blk, [row16[cc], v1])
            return carry
        lax.fori_loop(0, D, row_body, 0)

    fire_in(0, 0)

    def pair_body(kk, carry):
        for b in (0, 1):
            k = kk * 2 + b

            @pl.when(k + 1 < ITERS_PER_W)
            def _():
                fire_in(k + 1, 1 - b)

            wait_in(k, b)

            @pl.when(k >= 2)
            def _():
                wait_out(k - 2, b)

            transpose(b)
            fire_out(k, b)
        return carry

    lax.fori_loop(0, ITERS_PER_W // 2, pair_body, 0)
    wait_out(ITERS_PER_W - 2, 0)
    wait_out(ITERS_PER_W - 1, 1)

    @pl.when(wid < 4)
    def _():
        j = EXTRA_BASE + wid
        off = pl.multiple_of(j * TBLK, TBLK)
        pltpu.sync_copy(tableT_hbm.at[pl.ds(0, D), pl.ds(off, TBLK)],
                        blk0)
        transpose(0)
        pltpu.sync_copy(ob0, out_hbm.at[pl.ds(j * D, D)])

    @pl.when(wid == NW - 1)
    def _():
        pltpu.sync_copy(tail_hbm, blk1)

        def rem_row(r, carry):
            v0 = jnp.full((LANES,), 2 * r, jnp.int32)
            v1 = jnp.full((LANES,), 2 * r + 1, jnp.int32)
            for cc in range(4):
                ob1[r, pl.ds(cc * LANES, LANES)] = plsc.load_gather(
                    blk1, [row16[cc], v0])
                ob1[r, pl.ds(D + cc * LANES, LANES)] = plsc.load_gather(
                    blk1, [row16[cc], v1])
            return carry
        lax.fori_loop(0, VREM // 2, rem_row, 0)
        pltpu.sync_copy(ob1.at[pl.ds(0, VREM // 2)],
                        out_hbm.at[pl.ds(NFULL * D, VREM // 2)])


def _sc_detile(tableT, tail_pad):
    kfn = pl.kernel(
        _sc_detile_body,
        out_type=jax.ShapeDtypeStruct((V // 2, 2 * D), jnp.float32),
        mesh=plsc.VectorSubcoreMesh(core_axis_name="c", subcore_axis_name="s"),
        compiler_params=pltpu.CompilerParams(use_tc_tiling_on_sc=True,
                                             needs_layout_passes=False),
        scratch_types=[
            pltpu.VMEM((D, TBLK), jnp.float32),
            pltpu.VMEM((D, TBLK), jnp.float32),
            pltpu.VMEM((D, 2 * D), jnp.float32),
            pltpu.VMEM((D, 2 * D), jnp.float32),
            pltpu.SemaphoreType.DMA,
            pltpu.SemaphoreType.DMA,
            pltpu.SemaphoreType.DMA,
            pltpu.SemaphoreType.DMA,
        ],
    )
    return kfn(tableT, tail_pad)


BS = 1024


def _mlp_body(sum_ref, mask_ref, w1t_ref, b1_ref, w2p_ref, b2p_ref, out_ref):
    cnt = jnp.sum(mask_ref[...].astype(jnp.float32), axis=1, keepdims=True)
    pooled = sum_ref[...] / (cnt + 1e-9)
    h = jnp.dot(pooled, w1t_ref[...], preferred_element_type=jnp.float32)
    h = jnp.maximum(h + b1_ref[...], 0.0)
    out_ref[...] = (jnp.dot(h, w2p_ref[...], preferred_element_type=jnp.float32)
                    + b2p_ref[...])


def _mlp(pooled_sums, identity_mask, W1, b1, W2, b2):
    w1t = W1.T
    b1r = b1.reshape(1, H)
    w2p = jnp.zeros((H, 128), jnp.float32).at[:, :C].set(W2.T)
    b2p = jnp.zeros((1, 128), jnp.float32).at[0, :C].set(b2)
    out_pad = pl.pallas_call(
        _mlp_body,
        grid=(B // BS,),
        in_specs=[
            pl.BlockSpec((BS, D), lambda i: (i, 0)),
            pl.BlockSpec((BS, L), lambda i: (i, 0)),
            pl.BlockSpec((D, H), lambda i: (0, 0)),
            pl.BlockSpec((1, H), lambda i: (0, 0)),
            pl.BlockSpec((H, 128), lambda i: (0, 0)),
            pl.BlockSpec((1, 128), lambda i: (0, 0)),
        ],
        out_specs=pl.BlockSpec((BS, 128), lambda i: (i, 0)),
        out_shape=jax.ShapeDtypeStruct((B, 128), jnp.float32),
    )(pooled_sums, identity_mask, w1t, b1r, w2p, b2p)
    return out_pad[:, :C]


def kernel(input_ids, identity_mask, table, W1, b1, W2, b2):
    ids_flat = input_ids.reshape(B * L)
    mask_flat = identity_mask.reshape(B * L)
    tail_pad = jnp.pad(table[NFULL * TBLK:].T, ((0, 0), (0, TBLK - VREM)))
    table_lin = _sc_detile(table.T, tail_pad).reshape(V, D)
    pooled_sums = _sc_pool(ids_flat, mask_flat, table_lin)
    return _mlp(pooled_sums, identity_mask, W1, b1, W2, b2)


# speedup vs baseline: 3.0763x; 3.0763x over previous
import functools

import jax
import jax.numpy as jnp
from jax import lax
from jax.experimental import pallas as pl
from jax.experimental.pallas import tpu as pltpu
from jax.experimental.pallas import tpu_sc as plsc

B, L = 16384, 50
V, D = 1000000, 64
H, C = 64, 2

NW = 32
SEQ_PER_W = B // NW
S_CHUNK = 8
CT = S_CHUNK * L
NCH = SEQ_PER_W // S_CHUNK
GATHER_SPLITS = [(0, 128), (128, 128), (256, 128), (384, 16)]
LANES = 16


def _sc_pool_body(ids_hbm, mask_hbm, table_hbm, out_hbm,
                  idx0, idx1, y0, y1, m0, m1, rows0, rows1, wbuf, outv,
                  sem0, sem1):
    c = lax.axis_index("c")
    s = lax.axis_index("s")
    wid = s * 2 + c
    tok_base = wid * (SEQ_PER_W * L)
    seq_base = wid * SEQ_PER_W

    idx = [idx0, idx1]
    ybf = [y0, y1]
    msk = [m0, m1]
    rows = [rows0, rows1]
    sems = [sem0, sem1]

    def load_and_fire(k, b):
        tb = tok_base + k * CT
        pltpu.sync_copy(ids_hbm.at[pl.ds(tb, CT)], idx[b])
        pltpu.sync_copy(mask_hbm.at[pl.ds(tb, CT)], msk[b])
        for t in range(CT // LANES):
            x16 = idx[b][pl.ds(t * LANES, LANES)]
            r16 = x16 & (VB - 1)
            ybf[b][pl.ds(t * LANES, LANES)] = (
                (x16 & ~(VB - 1))
                | ((r16 & (VB // 2 - 1)) << 1)
                | (r16 >> 11))
        for (o, n) in GATHER_SPLITS:
            pltpu.async_copy(table_hbm.at[ybf[b].at[pl.ds(o, n)]],
                             rows[b].at[pl.ds(o, n)], sems[b])

    def wait_gathers(b):
        for (o, n) in GATHER_SPLITS:
            pltpu.make_async_copy(table_hbm.at[ybf[b].at[pl.ds(o, n)]],
                                  rows[b].at[pl.ds(o, n)], sems[b]).wait()

    def compute_weights(b):
        for t in range(CT // LANES):
            ids16 = idx[b][pl.ds(t * LANES, LANES)]
            mm16 = msk[b][pl.ds(t * LANES, LANES)]
            keep = (ids16 != 0) & (mm16 != 0)
            wbuf[pl.ds(t * LANES, LANES)] = jnp.where(keep, 1.0, 0.0)

    def accumulate(k, b):
        def seq_body(si, carry):
            tb = si * L
            wch = [wbuf[pl.ds(tb, LANES)],
                   wbuf[pl.ds(tb + 16, LANES)],
                   wbuf[pl.ds(tb + 32, LANES)],
                   wbuf[pl.ds(tb + 34, LANES)]]
            acc = [jnp.zeros((LANES,), jnp.float32) for _ in range(4)]
            for l in range(L):
                w = wch[l // 16][l % 16] if l < 48 else wch[3][l - 34]
                for j in range(4):
                    acc[j] = acc[j] + w * rows[b][tb + l, pl.ds(j * LANES, LANES)]
            for j in range(4):
                outv[si, pl.ds(j * LANES, LANES)] = acc[j]
            return carry
        lax.fori_loop(0, S_CHUNK, seq_body, 0)
        pltpu.sync_copy(outv, out_hbm.at[pl.ds(seq_base + k * S_CHUNK, S_CHUNK)])

    load_and_fire(0, 0)

    def pair_body(kk, carry):
        for b in (0, 1):
            k = kk * 2 + b

            @pl.when(k + 1 < NCH)
            def _():
                load_and_fire(k + 1, 1 - b)

            compute_weights(b)
            wait_gathers(b)
            accumulate(k, b)
        return carry

    lax.fori_loop(0, NCH // 2, pair_body, 0)


@functools.partial(jax.jit, static_argnames=())
def _sc_pool(ids_flat, mask_flat, table):
    kfn = pl.kernel(
        _sc_pool_body,
        out_type=jax.ShapeDtypeStruct((B, D), jnp.float32),
        mesh=plsc.VectorSubcoreMesh(core_axis_name="c", subcore_axis_name="s"),
        compiler_params=pltpu.CompilerParams(use_tc_tiling_on_sc=False),
        scratch_types=[
            pltpu.VMEM((CT,), jnp.int32),
            pltpu.VMEM((CT,), jnp.int32),
            pltpu.VMEM((CT,), jnp.int32),
            pltpu.VMEM((CT,), jnp.int32),
            pltpu.VMEM((CT,), jnp.int32),
            pltpu.VMEM((CT,), jnp.int32),
            pltpu.VMEM((CT, D), jnp.float32),
            pltpu.VMEM((CT, D), jnp.float32),
            pltpu.VMEM((CT,), jnp.float32),
            pltpu.VMEM((S_CHUNK, D), jnp.float32),
            pltpu.SemaphoreType.DMA,
            pltpu.SemaphoreType.DMA,
        ],
    )
    return kfn(ids_flat, mask_flat, table)


TBLK = 128
NFULL = V // TBLK
VREM = V - NFULL * TBLK
ITERS_PER_W = NFULL // NW
EXTRA_BASE = ITERS_PER_W * NW


def _sc_detile_body(tableT_hbm, tail_hbm, out_hbm, blk0, blk1, ob0, ob1,
                    isem0, isem1, osem0, osem1):
    c = lax.axis_index("c")
    s = lax.axis_index("s")
    wid = s * 2 + c

    blks = [blk0, blk1]
    obs = [ob0, ob1]
    isems = [isem0, isem1]
    osems = [osem0, osem1]

    row16 = [lax.iota(jnp.int32, LANES) + cc * LANES for cc in range(4)]

    def fire_in(k, b):
        j = wid + NW * k
        off = pl.multiple_of(j * TBLK, TBLK)
        for i in range(D // 8):
            pltpu.async_copy(
                tableT_hbm.at[pl.ds(8 * i, 8), pl.ds(off, TBLK)],
                blks[b].at[pl.ds(8 * i, 8)], isems[b])

    def wait_in(k, b):
        j = wid + NW * k
        off = pl.multiple_of(j * TBLK, TBLK)
        for i in range(D // 8):
            pltpu.make_async_copy(
                tableT_hbm.at[pl.ds(8 * i, 8), pl.ds(off, TBLK)],
                blks[b].at[pl.ds(8 * i, 8)], isems[b]).wait()

    def fire_out(k, b):
        j = wid + NW * k
        pltpu.async_copy(obs[b], out_hbm.at[pl.ds(j * D, D)], osems[b])

    def wait_out(k, b):
        j = wid + NW * k
        pltpu.make_async_copy(
            obs[b], out_hbm.at[pl.ds(j * D, D)], osems[b]).wait()

    def transpose(b):
        blk, ob = blks[b], obs[b]

        @plsc.parallel_loop(0, D, unroll=4)
        def _(r):
            v0 = jnp.full((LANES,), 2 * r, jnp.int32)
            v1 = jnp.full((LANES,), 2 * r + 1, jnp.int32)
            for cc in range(4):
                ob[r, pl.ds(cc * LANES, LANES)] = plsc.load_gather(
                    blk, [row16[cc], v0])
                ob[r, pl.ds(D + cc * LANES, LANES)] = plsc.load_gather(
                    blk, [row16[cc], v1])

    fire_in(0, 0)

    def pair_body(kk, carry):
        for b in (0, 1):
            k = kk * 2 + b

            @pl.when(k + 1 < ITERS_PER_W)
            def _():
                fire_in(k + 1, 1 - b)

            wait_in(k, b)

            @pl.when(k >= 2)
            def _():
                wait_out(k - 2, b)

            transpose(b)
            fire_out(k, b)
        return carry

    lax.fori_loop(0, ITERS_PER_W // 2, pair_body, 0)
    wait_out(ITERS_PER_W - 2, 0)
    wait_out(ITERS_PER_W - 1, 1)

    @pl.when(wid < 4)
    def _():
        j = EXTRA_BASE + wid
        off = pl.multiple_of(j * TBLK, TBLK)
        pltpu.sync_copy(tableT_hbm.at[pl.ds(0, D), pl.ds(off, TBLK)],
                        blk0)
        transpose(0)
        pltpu.sync_copy(ob0, out_hbm.at[pl.ds(j * D, D)])

    @pl.when(wid == NW - 1)
    def _():
        pltpu.sync_copy(tail_hbm, blk1)

        def rem_row(r, carry):
            v0 = jnp.full((LANES,), 2 * r, jnp.int32)
            v1 = jnp.full((LANES,), 2 * r + 1, jnp.int32)
            for cc in range(4):
                ob1[r, pl.ds(cc * LANES, LANES)] = plsc.load_gather(
                    blk1, [row16[cc], v0])
                ob1[r, pl.ds(D + cc * LANES, LANES)] = plsc.load_gather(
                    blk1, [row16[cc], v1])
            return carry
        lax.fori_loop(0, VREM // 2, rem_row, 0)
        pltpu.sync_copy(ob1.at[pl.ds(0, VREM // 2)],
                        out_hbm.at[pl.ds(NFULL * D, VREM // 2)])


def _sc_detile(tableT, tail_pad):
    kfn = pl.kernel(
        _sc_detile_body,
        out_type=jax.ShapeDtypeStruct((V // 2, 2 * D), jnp.float32),
        mesh=plsc.VectorSubcoreMesh(core_axis_name="c", subcore_axis_name="s"),
        compiler_params=pltpu.CompilerParams(use_tc_tiling_on_sc=True,
                                             needs_layout_passes=False),
        scratch_types=[
            pltpu.VMEM((D, TBLK), jnp.float32),
            pltpu.VMEM((D, TBLK), jnp.float32),
            pltpu.VMEM((D, 2 * D), jnp.float32),
            pltpu.VMEM((D, 2 * D), jnp.float32),
            pltpu.SemaphoreType.DMA,
            pltpu.SemaphoreType.DMA,
            pltpu.SemaphoreType.DMA,
            pltpu.SemaphoreType.DMA,
        ],
    )
    return kfn(tableT, tail_pad)


VB = 4096
TC_GRID = -(-V // VB)
VP = TC_GRID * VB


def _tc_detile_body(xT_ref, out_ref):
    xt = xT_ref[...].T
    out_ref[...] = jnp.concatenate([xt[:VB // 2, :], xt[VB // 2:, :]], axis=1)


def _tc_detile(tableT):
    return pl.pallas_call(
        _tc_detile_body,
        grid=(TC_GRID,),
        in_specs=[pl.BlockSpec((D, VB), lambda g: (0, g))],
        out_specs=pl.BlockSpec((VB // 2, 2 * D), lambda g: (g, 0)),
        out_shape=jax.ShapeDtypeStruct((VP // 2, 2 * D), jnp.float32),
    )(tableT)


BS = 1024


def _mlp_body(sum_ref, mask_ref, w1t_ref, b1_ref, w2p_ref, b2p_ref, out_ref):
    cnt = jnp.sum(mask_ref[...].astype(jnp.float32), axis=1, keepdims=True)
    pooled = sum_ref[...] / (cnt + 1e-9)
    h = jnp.dot(pooled, w1t_ref[...], preferred_element_type=jnp.float32)
    h = jnp.maximum(h + b1_ref[...], 0.0)
    out_ref[...] = (jnp.dot(h, w2p_ref[...], preferred_element_type=jnp.float32)
                    + b2p_ref[...])


def _mlp(pooled_sums, identity_mask, W1, b1, W2, b2):
    w1t = W1.T
    b1r = b1.reshape(1, H)
    w2p = jnp.zeros((H, 128), jnp.float32).at[:, :C].set(W2.T)
    b2p = jnp.zeros((1, 128), jnp.float32).at[0, :C].set(b2)
    out_pad = pl.pallas_call(
        _mlp_body,
        grid=(B // BS,),
        in_specs=[
            pl.BlockSpec((BS, D), lambda i: (i, 0)),
            pl.BlockSpec((BS, L), lambda i: (i, 0)),
            pl.BlockSpec((D, H), lambda i: (0, 0)),
            pl.BlockSpec((1, H), lambda i: (0, 0)),
            pl.BlockSpec((H, 128), lambda i: (0, 0)),
            pl.BlockSpec((1, 128), lambda i: (0, 0)),
        ],
        out_specs=pl.BlockSpec((BS, 128), lambda i: (i, 0)),
        out_shape=jax.ShapeDtypeStruct((B, 128), jnp.float32),
    )(pooled_sums, identity_mask, w1t, b1r, w2p, b2p)
    return out_pad[:, :C]


def kernel(input_ids, identity_mask, table, W1, b1, W2, b2):
    ids_flat = input_ids.reshape(B * L)
    mask_flat = identity_mask.reshape(B * L)
    table_lin = _tc_detile(table.T).reshape(VP, D)
    pooled_sums = _sc_pool(ids_flat, mask_flat, table_lin)
    return _mlp(pooled_sums, identity_mask, W1, b1, W2, b2)


# speedup vs baseline: 3.0814x; 1.0016x over previous
import functools

import jax
import jax.numpy as jnp
from jax import lax
from jax.experimental import pallas as pl
from jax.experimental.pallas import tpu as pltpu
from jax.experimental.pallas import tpu_sc as plsc

B, L = 16384, 50
V, D = 1000000, 64
H, C = 64, 2

NW = 32
SEQ_PER_W = B // NW
S_CHUNK = 8
CT = S_CHUNK * L
NCH = SEQ_PER_W // S_CHUNK
GATHER_SPLITS = [(0, 128), (128, 128), (256, 128), (384, 16)]
LANES = 16


def _sc_pool_body(ids_hbm, mask_hbm, table_hbm, out_hbm,
                  idx0, idx1, y0, y1, m0, m1, rows0, rows1, wbuf, outv,
                  sem0, sem1):
    c = lax.axis_index("c")
    s = lax.axis_index("s")
    wid = s * 2 + c
    tok_base = wid * (SEQ_PER_W * L)
    seq_base = wid * SEQ_PER_W

    idx = [idx0, idx1]
    ybf = [y0, y1]
    msk = [m0, m1]
    rows = [rows0, rows1]
    sems = [sem0, sem1]

    def load_and_fire(k, b):
        tb = tok_base + k * CT
        pltpu.sync_copy(ids_hbm.at[pl.ds(tb, CT)], idx[b])
        pltpu.sync_copy(mask_hbm.at[pl.ds(tb, CT)], msk[b])
        for t in range(CT // LANES):
            x16 = idx[b][pl.ds(t * LANES, LANES)]
            r16 = x16 & (VB - 1)
            ybf[b][pl.ds(t * LANES, LANES)] = (
                (x16 & ~(VB - 1))
                | ((r16 & (VB // 2 - 1)) << 1)
                | (r16 >> 11))
        for (o, n) in GATHER_SPLITS:
            pltpu.async_copy(table_hbm.at[ybf[b].at[pl.ds(o, n)]],
                             rows[b].at[pl.ds(o, n)], sems[b])

    def wait_gathers(b):
        for (o, n) in GATHER_SPLITS:
            pltpu.make_async_copy(table_hbm.at[ybf[b].at[pl.ds(o, n)]],
                                  rows[b].at[pl.ds(o, n)], sems[b]).wait()

    def compute_weights(b):
        for t in range(CT // LANES):
            ids16 = idx[b][pl.ds(t * LANES, LANES)]
            mm16 = msk[b][pl.ds(t * LANES, LANES)]
            keep = (ids16 != 0) & (mm16 != 0)
            wbuf[pl.ds(t * LANES, LANES)] = jnp.where(keep, 1.0, 0.0)

    def accumulate(k, b):
        def seq_body(si, carry):
            tb = si * L
            wch = [wbuf[pl.ds(tb, LANES)],
                   wbuf[pl.ds(tb + 16, LANES)],
                   wbuf[pl.ds(tb + 32, LANES)],
                   wbuf[pl.ds(tb + 34, LANES)]]
            acc = [jnp.zeros((LANES,), jnp.float32) for _ in range(4)]
            for l in range(L):
                w = wch[l // 16][l % 16] if l < 48 else wch[3][l - 34]
                for j in range(4):
                    acc[j] = acc[j] + w * rows[b][tb + l, pl.ds(j * LANES, LANES)]
            for j in range(4):
                outv[si, pl.ds(j * LANES, LANES)] = acc[j]
            return carry
        lax.fori_loop(0, S_CHUNK, seq_body, 0)
        pltpu.sync_copy(outv, out_hbm.at[pl.ds(seq_base + k * S_CHUNK, S_CHUNK)])

    load_and_fire(0, 0)

    def pair_body(kk, carry):
        for b in (0, 1):
            k = kk * 2 + b

            @pl.when(k + 1 < NCH)
            def _():
                load_and_fire(k + 1, 1 - b)

            compute_weights(b)
            wait_gathers(b)
            accumulate(k, b)
        return carry

    lax.fori_loop(0, NCH // 2, pair_body, 0)


@functools.partial(jax.jit, static_argnames=())
def _sc_pool(ids_flat, mask_flat, table):
    kfn = pl.kernel(
        _sc_pool_body,
        out_type=jax.ShapeDtypeStruct((B, D), jnp.float32),
        mesh=plsc.VectorSubcoreMesh(core_axis_name="c", subcore_axis_name="s"),
        compiler_params=pltpu.CompilerParams(use_tc_tiling_on_sc=False),
        scratch_types=[
            pltpu.VMEM((CT,), jnp.int32),
            pltpu.VMEM((CT,), jnp.int32),
            pltpu.VMEM((CT,), jnp.int32),
            pltpu.VMEM((CT,), jnp.int32),
            pltpu.VMEM((CT,), jnp.int32),
            pltpu.VMEM((CT,), jnp.int32),
            pltpu.VMEM((CT, D), jnp.float32),
            pltpu.VMEM((CT, D), jnp.float32),
            pltpu.VMEM((CT,), jnp.float32),
            pltpu.VMEM((S_CHUNK, D), jnp.float32),
            pltpu.SemaphoreType.DMA,
            pltpu.SemaphoreType.DMA,
        ],
    )
    return kfn(ids_flat, mask_flat, table)


TBLK = 128
NFULL = V // TBLK
VREM = V - NFULL * TBLK
ITERS_PER_W = NFULL // NW
EXTRA_BASE = ITERS_PER_W * NW


def _sc_detile_body(tableT_hbm, tail_hbm, out_hbm, blk0, blk1, ob0, ob1,
                    isem0, isem1, osem0, osem1):
    c = lax.axis_index("c")
    s = lax.axis_index("s")
    wid = s * 2 + c

    blks = [blk0, blk1]
    obs = [ob0, ob1]
    isems = [isem0, isem1]
    osems = [osem0, osem1]

    row16 = [lax.iota(jnp.int32, LANES) + cc * LANES for cc in range(4)]

    def fire_in(k, b):
        j = wid + NW * k
        off = pl.multiple_of(j * TBLK, TBLK)
        for i in range(D // 8):
            pltpu.async_copy(
                tableT_hbm.at[pl.ds(8 * i, 8), pl.ds(off, TBLK)],
                blks[b].at[pl.ds(8 * i, 8)], isems[b])

    def wait_in(k, b):
        j = wid + NW * k
        off = pl.multiple_of(j * TBLK, TBLK)
        for i in range(D // 8):
            pltpu.make_async_copy(
                tableT_hbm.at[pl.ds(8 * i, 8), pl.ds(off, TBLK)],
                blks[b].at[pl.ds(8 * i, 8)], isems[b]).wait()

    def fire_out(k, b):
        j = wid + NW * k
        pltpu.async_copy(obs[b], out_hbm.at[pl.ds(j * D, D)], osems[b])

    def wait_out(k, b):
        j = wid + NW * k
        pltpu.make_async_copy(
            obs[b], out_hbm.at[pl.ds(j * D, D)], osems[b]).wait()

    def transpose(b):
        blk, ob = blks[b], obs[b]

        @plsc.parallel_loop(0, D, unroll=4)
        def _(r):
            v0 = jnp.full((LANES,), 2 * r, jnp.int32)
            v1 = jnp.full((LANES,), 2 * r + 1, jnp.int32)
            for cc in range(4):
                ob[r, pl.ds(cc * LANES, LANES)] = plsc.load_gather(
                    blk, [row16[cc], v0])
                ob[r, pl.ds(D + cc * LANES, LANES)] = plsc.load_gather(
                    blk, [row16[cc], v1])

    fire_in(0, 0)

    def pair_body(kk, carry):
        for b in (0, 1):
            k = kk * 2 + b

            @pl.when(k + 1 < ITERS_PER_W)
            def _():
                fire_in(k + 1, 1 - b)

            wait_in(k, b)

            @pl.when(k >= 2)
            def _():
                wait_out(k - 2, b)

            transpose(b)
            fire_out(k, b)
        return carry

    lax.fori_loop(0, ITERS_PER_W // 2, pair_body, 0)
    wait_out(ITERS_PER_W - 2, 0)
    wait_out(ITERS_PER_W - 1, 1)

    @pl.when(wid < 4)
    def _():
        j = EXTRA_BASE + wid
        off = pl.multiple_of(j * TBLK, TBLK)
        pltpu.sync_copy(tableT_hbm.at[pl.ds(0, D), pl.ds(off, TBLK)],
                        blk0)
        transpose(0)
        pltpu.sync_copy(ob0, out_hbm.at[pl.ds(j * D, D)])

    @pl.when(wid == NW - 1)
    def _():
        pltpu.sync_copy(tail_hbm, blk1)

        def rem_row(r, carry):
            v0 = jnp.full((LANES,), 2 * r, jnp.int32)
            v1 = jnp.full((LANES,), 2 * r + 1, jnp.int32)
            for cc in range(4):
                ob1[r, pl.ds(cc * LANES, LANES)] = plsc.load_gather(
                    blk1, [row16[cc], v0])
                ob1[r, pl.ds(D + cc * LANES, LANES)] = plsc.load_gather(
                    blk1, [row16[cc], v1])
            return carry
        lax.fori_loop(0, VREM // 2, rem_row, 0)
        pltpu.sync_copy(ob1.at[pl.ds(0, VREM // 2)],
                        out_hbm.at[pl.ds(NFULL * D, VREM // 2)])


def _sc_detile(tableT, tail_pad):
    kfn = pl.kernel(
        _sc_detile_body,
        out_type=jax.ShapeDtypeStruct((V // 2, 2 * D), jnp.float32),
        mesh=plsc.VectorSubcoreMesh(core_axis_name="c", subcore_axis_name="s"),
        compiler_params=pltpu.CompilerParams(use_tc_tiling_on_sc=True,
                                             needs_layout_passes=False),
        scratch_types=[
            pltpu.VMEM((D, TBLK), jnp.float32),
            pltpu.VMEM((D, TBLK), jnp.float32),
            pltpu.VMEM((D, 2 * D), jnp.float32),
            pltpu.VMEM((D, 2 * D), jnp.float32),
            pltpu.SemaphoreType.DMA,
            pltpu.SemaphoreType.DMA,
            pltpu.SemaphoreType.DMA,
            pltpu.SemaphoreType.DMA,
        ],
    )
    return kfn(tableT, tail_pad)


VB = 4096
TC_GRID = -(-V // VB)
VP = TC_GRID * VB


def _tc_detile_body(xT_ref, out_ref):
    xt = xT_ref[...].T
    out_ref[...] = jnp.concatenate([xt[:VB // 2, :], xt[VB // 2:, :]], axis=1)


def _tc_detile(tableT):
    return pl.pallas_call(
        _tc_detile_body,
        grid=(TC_GRID,),
        in_specs=[pl.BlockSpec((D, VB), lambda g: (0, g))],
        out_specs=pl.BlockSpec((VB // 2, 2 * D), lambda g: (g, 0)),
        out_shape=jax.ShapeDtypeStruct((VP // 2, 2 * D), jnp.float32),
        compiler_params=pltpu.CompilerParams(
            dimension_semantics=("parallel",)),
    )(tableT)


BS = 1024


def _mlp_body(sum_ref, mask_ref, w1t_ref, b1_ref, w2p_ref, b2p_ref, out_ref):
    cnt = jnp.sum(mask_ref[...].astype(jnp.float32), axis=1, keepdims=True)
    pooled = sum_ref[...] / (cnt + 1e-9)
    h = jnp.dot(pooled, w1t_ref[...], preferred_element_type=jnp.float32)
    h = jnp.maximum(h + b1_ref[...], 0.0)
    out_ref[...] = (jnp.dot(h, w2p_ref[...], preferred_element_type=jnp.float32)
                    + b2p_ref[...])


def _mlp(pooled_sums, identity_mask, W1, b1, W2, b2):
    w1t = W1.T
    b1r = b1.reshape(1, H)
    w2p = jnp.zeros((H, 128), jnp.float32).at[:, :C].set(W2.T)
    b2p = jnp.zeros((1, 128), jnp.float32).at[0, :C].set(b2)
    out_pad = pl.pallas_call(
        _mlp_body,
        grid=(B // BS,),
        in_specs=[
            pl.BlockSpec((BS, D), lambda i: (i, 0)),
            pl.BlockSpec((BS, L), lambda i: (i, 0)),
            pl.BlockSpec((D, H), lambda i: (0, 0)),
            pl.BlockSpec((1, H), lambda i: (0, 0)),
            pl.BlockSpec((H, 128), lambda i: (0, 0)),
            pl.BlockSpec((1, 128), lambda i: (0, 0)),
        ],
        out_specs=pl.BlockSpec((BS, 128), lambda i: (i, 0)),
        out_shape=jax.ShapeDtypeStruct((B, 128), jnp.float32),
    )(pooled_sums, identity_mask, w1t, b1r, w2p, b2p)
    return out_pad[:, :C]


def kernel(input_ids, identity_mask, table, W1, b1, W2, b2):
    ids_flat = input_ids.reshape(B * L)
    mask_flat = identity_mask.reshape(B * L)
    table_lin = _tc_detile(table.T).reshape(VP, D)
    pooled_sums = _sc_pool(ids_flat, mask_flat, table_lin)
    return _mlp(pooled_sums, identity_mask, W1, b1, W2, b2)


# speedup vs baseline: 3.5135x; 1.1403x over previous
import functools

import jax
import jax.numpy as jnp
from jax import lax
from jax.experimental import pallas as pl
from jax.experimental.pallas import tpu as pltpu
from jax.experimental.pallas import tpu_sc as plsc

B, L = 16384, 50
V, D = 1000000, 64
H, C = 64, 2

NW = 32
SEQ_PER_W = B // NW
S_CHUNK = 8
CT = S_CHUNK * L
NCH = SEQ_PER_W // S_CHUNK
GATHER_SPLITS = [(0, 128), (128, 128), (256, 128), (384, 16)]
LANES = 16


def _sc_pool_body(ids_hbm, mask_hbm, table_hbm, out_hbm,
                  idx0, idx1, y0, y1, m0, m1, rows0, rows1, wbuf, outv,
                  sem0, sem1):
    c = lax.axis_index("c")
    s = lax.axis_index("s")
    wid = s * 2 + c
    tok_base = wid * (SEQ_PER_W * L)
    seq_base = wid * SEQ_PER_W

    idx = [idx0, idx1]
    ybf = [y0, y1]
    msk = [m0, m1]
    rows = [rows0, rows1]
    sems = [sem0, sem1]

    def load_and_fire(k, b):
        tb = tok_base + k * CT
        pltpu.sync_copy(ids_hbm.at[pl.ds(tb, CT)], idx[b])
        pltpu.sync_copy(mask_hbm.at[pl.ds(tb, CT)], msk[b])
        for t in range(CT // LANES):
            x16 = idx[b][pl.ds(t * LANES, LANES)]
            r16 = x16 & (VB - 1)
            ybf[b][pl.ds(t * LANES, LANES)] = (
                (x16 & ~(VB - 1))
                | ((r16 & (VB // 2 - 1)) << 1)
                | (r16 >> VB_SH))
        for (o, n) in GATHER_SPLITS:
            pltpu.async_copy(table_hbm.at[ybf[b].at[pl.ds(o, n)]],
                             rows[b].at[pl.ds(o, n)], sems[b])

    def wait_gathers(b):
        for (o, n) in GATHER_SPLITS:
            pltpu.make_async_copy(table_hbm.at[ybf[b].at[pl.ds(o, n)]],
                                  rows[b].at[pl.ds(o, n)], sems[b]).wait()

    def compute_weights(b):
        for t in range(CT // LANES):
            ids16 = idx[b][pl.ds(t * LANES, LANES)]
            mm16 = msk[b][pl.ds(t * LANES, LANES)]
            keep = (ids16 != 0) & (mm16 != 0)
            wbuf[pl.ds(t * LANES, LANES)] = jnp.where(keep, 1.0, 0.0)

    def accumulate(k, b):
        def seq_body(si, carry):
            tb = si * L
            wch = [wbuf[pl.ds(tb, LANES)],
                   wbuf[pl.ds(tb + 16, LANES)],
                   wbuf[pl.ds(tb + 32, LANES)],
                   wbuf[pl.ds(tb + 34, LANES)]]
            acc = [jnp.zeros((LANES,), jnp.float32) for _ in range(4)]
            for l in range(L):
                w = wch[l // 16][l % 16] if l < 48 else wch[3][l - 34]
                for j in range(4):
                    acc[j] = acc[j] + w * rows[b][tb + l, pl.ds(j * LANES, LANES)]
            for j in range(4):
                outv[si, pl.ds(j * LANES, LANES)] = acc[j]
            return carry
        lax.fori_loop(0, S_CHUNK, seq_body, 0)
        pltpu.sync_copy(outv, out_hbm.at[pl.ds(seq_base + k * S_CHUNK, S_CHUNK)])

    load_and_fire(0, 0)

    def pair_body(kk, carry):
        for b in (0, 1):
            k = kk * 2 + b

            @pl.when(k + 1 < NCH)
            def _():
                load_and_fire(k + 1, 1 - b)

            compute_weights(b)
            wait_gathers(b)
            accumulate(k, b)
        return carry

    lax.fori_loop(0, NCH // 2, pair_body, 0)


@functools.partial(jax.jit, static_argnames=())
def _sc_pool(ids_flat, mask_flat, table):
    kfn = pl.kernel(
        _sc_pool_body,
        out_type=jax.ShapeDtypeStruct((B, D), jnp.float32),
        mesh=plsc.VectorSubcoreMesh(core_axis_name="c", subcore_axis_name="s"),
        compiler_params=pltpu.CompilerParams(use_tc_tiling_on_sc=False),
        scratch_types=[
            pltpu.VMEM((CT,), jnp.int32),
            pltpu.VMEM((CT,), jnp.int32),
            pltpu.VMEM((CT,), jnp.int32),
            pltpu.VMEM((CT,), jnp.int32),
            pltpu.VMEM((CT,), jnp.int32),
            pltpu.VMEM((CT,), jnp.int32),
            pltpu.VMEM((CT, D), jnp.float32),
            pltpu.VMEM((CT, D), jnp.float32),
            pltpu.VMEM((CT,), jnp.float32),
            pltpu.VMEM((S_CHUNK, D), jnp.float32),
            pltpu.SemaphoreType.DMA,
            pltpu.SemaphoreType.DMA,
        ],
    )
    return kfn(ids_flat, mask_flat, table)


TBLK = 128
NFULL = V // TBLK
VREM = V - NFULL * TBLK
ITERS_PER_W = NFULL // NW
EXTRA_BASE = ITERS_PER_W * NW


def _sc_detile_body(tableT_hbm, tail_hbm, out_hbm, blk0, blk1, ob0, ob1,
                    isem0, isem1, osem0, osem1):
    c = lax.axis_index("c")
    s = lax.axis_index("s")
    wid = s * 2 + c

    blks = [blk0, blk1]
    obs = [ob0, ob1]
    isems = [isem0, isem1]
    osems = [osem0, osem1]

    row16 = [lax.iota(jnp.int32, LANES) + cc * LANES for cc in range(4)]

    def fire_in(k, b):
        j = wid + NW * k
        off = pl.multiple_of(j * TBLK, TBLK)
        for i in range(D // 8):
            pltpu.async_copy(
                tableT_hbm.at[pl.ds(8 * i, 8), pl.ds(off, TBLK)],
                blks[b].at[pl.ds(8 * i, 8)], isems[b])

    def wait_in(k, b):
        j = wid + NW * k
        off = pl.multiple_of(j * TBLK, TBLK)
        for i in range(D // 8):
            pltpu.make_async_copy(
                tableT_hbm.at[pl.ds(8 * i, 8), pl.ds(off, TBLK)],
                blks[b].at[pl.ds(8 * i, 8)], isems[b]).wait()

    def fire_out(k, b):
        j = wid + NW * k
        pltpu.async_copy(obs[b], out_hbm.at[pl.ds(j * D, D)], osems[b])

    def wait_out(k, b):
        j = wid + NW * k
        pltpu.make_async_copy(
            obs[b], out_hbm.at[pl.ds(j * D, D)], osems[b]).wait()

    def transpose(b):
        blk, ob = blks[b], obs[b]

        @plsc.parallel_loop(0, D, unroll=4)
        def _(r):
            v0 = jnp.full((LANES,), 2 * r, jnp.int32)
            v1 = jnp.full((LANES,), 2 * r + 1, jnp.int32)
            for cc in range(4):
                ob[r, pl.ds(cc * LANES, LANES)] = plsc.load_gather(
                    blk, [row16[cc], v0])
                ob[r, pl.ds(D + cc * LANES, LANES)] = plsc.load_gather(
                    blk, [row16[cc], v1])

    fire_in(0, 0)

    def pair_body(kk, carry):
        for b in (0, 1):
            k = kk * 2 + b

            @pl.when(k + 1 < ITERS_PER_W)
            def _():
                fire_in(k + 1, 1 - b)

            wait_in(k, b)

            @pl.when(k >= 2)
            def _():
                wait_out(k - 2, b)

            transpose(b)
            fire_out(k, b)
        return carry

    lax.fori_loop(0, ITERS_PER_W // 2, pair_body, 0)
    wait_out(ITERS_PER_W - 2, 0)
    wait_out(ITERS_PER_W - 1, 1)

    @pl.when(wid < 4)
    def _():
        j = EXTRA_BASE + wid
        off = pl.multiple_of(j * TBLK, TBLK)
        pltpu.sync_copy(tableT_hbm.at[pl.ds(0, D), pl.ds(off, TBLK)],
                        blk0)
        transpose(0)
        pltpu.sync_copy(ob0, out_hbm.at[pl.ds(j * D, D)])

    @pl.when(wid == NW - 1)
    def _():
        pltpu.sync_copy(tail_hbm, blk1)

        def rem_row(r, carry):
            v0 = jnp.full((LANES,), 2 * r, jnp.int32)
            v1 = jnp.full((LANES,), 2 * r + 1, jnp.int32)
            for cc in range(4):
                ob1[r, pl.ds(cc * LANES, LANES)] = plsc.load_gather(
                    blk1, [row16[cc], v0])
                ob1[r, pl.ds(D + cc * LANES, LANES)] = plsc.load_gather(
                    blk1, [row16[cc], v1])
            return carry
        lax.fori_loop(0, VREM // 2, rem_row, 0)
        pltpu.sync_copy(ob1.at[pl.ds(0, VREM // 2)],
                        out_hbm.at[pl.ds(NFULL * D, VREM // 2)])


def _sc_detile(tableT, tail_pad):
    kfn = pl.kernel(
        _sc_detile_body,
        out_type=jax.ShapeDtypeStruct((V // 2, 2 * D), jnp.float32),
        mesh=plsc.VectorSubcoreMesh(core_axis_name="c", subcore_axis_name="s"),
        compiler_params=pltpu.CompilerParams(use_tc_tiling_on_sc=True,
                                             needs_layout_passes=False),
        scratch_types=[
            pltpu.VMEM((D, TBLK), jnp.float32),
            pltpu.VMEM((D, TBLK), jnp.float32),
            pltpu.VMEM((D, 2 * D), jnp.float32),
            pltpu.VMEM((D, 2 * D), jnp.float32),
            pltpu.SemaphoreType.DMA,
            pltpu.SemaphoreType.DMA,
            pltpu.SemaphoreType.DMA,
            pltpu.SemaphoreType.DMA,
        ],
    )
    return kfn(tableT, tail_pad)


VB = 8192
VB_SH = (VB // 2).bit_length() - 1
TC_GRID = -(-V // VB)
VP = TC_GRID * VB


def _tc_detile_body(xT_ref, out_ref):
    xt = xT_ref[...].T
    out_ref[...] = jnp.concatenate([xt[:VB // 2, :], xt[VB // 2:, :]], axis=1)


def _tc_detile(tableT):
    return pl.pallas_call(
        _tc_detile_body,
        grid=(TC_GRID,),
        in_specs=[pl.BlockSpec((D, VB), lambda g: (0, g))],
        out_specs=pl.BlockSpec((VB // 2, 2 * D), lambda g: (g, 0)),
        out_shape=jax.ShapeDtypeStruct((VP // 2, 2 * D), jnp.float32),
        compiler_params=pltpu.CompilerParams(
            dimension_semantics=("parallel",)),
    )(tableT)


BS = 1024


def _mlp_body(sum_ref, mask_ref, w1t_ref, b1_ref, w2p_ref, b2p_ref, out_ref):
    cnt = jnp.sum(mask_ref[...].astype(jnp.float32), axis=1, keepdims=True)
    pooled = sum_ref[...] / (cnt + 1e-9)
    h = jnp.dot(pooled, w1t_ref[...], preferred_element_type=jnp.float32)
    h = jnp.maximum(h + b1_ref[...], 0.0)
    out_ref[...] = (jnp.dot(h, w2p_ref[...], preferred_element_type=jnp.float32)
                    + b2p_ref[...])


def _mlp(pooled_sums, identity_mask, W1, b1, W2, b2):
    w1t = W1.T
    b1r = b1.reshape(1, H)
    w2p = jnp.zeros((H, 128), jnp.float32).at[:, :C].set(W2.T)
    b2p = jnp.zeros((1, 128), jnp.float32).at[0, :C].set(b2)
    out_pad = pl.pallas_call(
        _mlp_body,
        grid=(B // BS,),
        in_specs=[
            pl.BlockSpec((BS, D), lambda i: (i, 0)),
            pl.BlockSpec((BS, L), lambda i: (i, 0)),
            pl.BlockSpec((D, H), lambda i: (0, 0)),
            pl.BlockSpec((1, H), lambda i: (0, 0)),
            pl.BlockSpec((H, 128), lambda i: (0, 0)),
            pl.BlockSpec((1, 128), lambda i: (0, 0)),
        ],
        out_specs=pl.BlockSpec((BS, 128), lambda i: (i, 0)),
        out_shape=jax.ShapeDtypeStruct((B, 128), jnp.float32),
    )(pooled_sums, identity_mask, w1t, b1r, w2p, b2p)
    return out_pad[:, :C]


def kernel(input_ids, identity_mask, table, W1, b1, W2, b2):
    ids_flat = input_ids.reshape(B * L)
    mask_flat = identity_mask.reshape(B * L)
    table_lin = _tc_detile(table.T).reshape(VP, D)
    pooled_sums = _sc_pool(ids_flat, mask_flat, table_lin)
    return _mlp(pooled_sums, identity_mask, W1, b1, W2, b2)


# speedup vs baseline: 3.7546x; 1.0686x over previous
import functools

import jax
import jax.numpy as jnp
from jax import lax
from jax.experimental import pallas as pl
from jax.experimental.pallas import tpu as pltpu
from jax.experimental.pallas import tpu_sc as plsc

B, L = 16384, 50
V, D = 1000000, 64
H, C = 64, 2

NW = 32
SEQ_PER_W = B // NW
S_CHUNK = 8
CT = S_CHUNK * L
NCH = SEQ_PER_W // S_CHUNK
GATHER_SPLITS = [(0, 128), (128, 128), (256, 128), (384, 16)]
LANES = 16


def _sc_pool_body(ids_hbm, mask_hbm, table_hbm, out_hbm,
                  idx0, idx1, y0, y1, m0, m1, rows0, rows1, wbuf, outv,
                  sem0, sem1):
    c = lax.axis_index("c")
    s = lax.axis_index("s")
    wid = s * 2 + c
    tok_base = wid * (SEQ_PER_W * L)
    seq_base = wid * SEQ_PER_W

    idx = [idx0, idx1]
    ybf = [y0, y1]
    msk = [m0, m1]
    rows = [rows0, rows1]
    sems = [sem0, sem1]

    def load_and_fire(k, b):
        tb = tok_base + k * CT
        pltpu.sync_copy(ids_hbm.at[pl.ds(tb, CT)], idx[b])
        pltpu.sync_copy(mask_hbm.at[pl.ds(tb, CT)], msk[b])
        for t in range(CT // LANES):
            x16 = idx[b][pl.ds(t * LANES, LANES)]
            r16 = x16 & (VB - 1)
            ybf[b][pl.ds(t * LANES, LANES)] = (
                (x16 & ~(VB - 1))
                | ((r16 & (VB // 2 - 1)) << 1)
                | (r16 >> VB_SH))
        for (o, n) in GATHER_SPLITS:
            pltpu.async_copy(table_hbm.at[ybf[b].at[pl.ds(o, n)]],
                             rows[b].at[pl.ds(o, n)], sems[b])

    def wait_gathers(b):
        for (o, n) in GATHER_SPLITS:
            pltpu.make_async_copy(table_hbm.at[ybf[b].at[pl.ds(o, n)]],
                                  rows[b].at[pl.ds(o, n)], sems[b]).wait()

    def compute_weights(b):
        for t in range(CT // LANES):
            ids16 = idx[b][pl.ds(t * LANES, LANES)]
            mm16 = msk[b][pl.ds(t * LANES, LANES)]
            keep = (ids16 != 0) & (mm16 != 0)
            wbuf[pl.ds(t * LANES, LANES)] = jnp.where(keep, 1.0, 0.0)

    def accumulate(k, b):
        def seq_body(si, carry):
            tb = si * L
            wch = [wbuf[pl.ds(tb, LANES)],
                   wbuf[pl.ds(tb + 16, LANES)],
                   wbuf[pl.ds(tb + 32, LANES)],
                   wbuf[pl.ds(tb + 34, LANES)]]
            acc = [jnp.zeros((LANES,), jnp.float32) for _ in range(4)]
            for l in range(L):
                w = wch[l // 16][l % 16] if l < 48 else wch[3][l - 34]
                for j in range(4):
                    acc[j] = acc[j] + w * rows[b][tb + l, pl.ds(j * LANES, LANES)]
            for j in range(4):
                outv[si, pl.ds(j * LANES, LANES)] = acc[j]
            return carry
        lax.fori_loop(0, S_CHUNK, seq_body, 0)
        pltpu.sync_copy(outv, out_hbm.at[pl.ds(seq_base + k * S_CHUNK, S_CHUNK)])

    load_and_fire(0, 0)

    def pair_body(kk, carry):
        for b in (0, 1):
            k = kk * 2 + b

            @pl.when(k + 1 < NCH)
            def _():
                load_and_fire(k + 1, 1 - b)

            compute_weights(b)
            wait_gathers(b)
            accumulate(k, b)
        return carry

    lax.fori_loop(0, NCH // 2, pair_body, 0)


@functools.partial(jax.jit, static_argnames=())
def _sc_pool(ids_flat, mask_flat, table):
    kfn = pl.kernel(
        _sc_pool_body,
        out_type=jax.ShapeDtypeStruct((B, D), jnp.float32),
        mesh=plsc.VectorSubcoreMesh(core_axis_name="c", subcore_axis_name="s"),
        compiler_params=pltpu.CompilerParams(use_tc_tiling_on_sc=False),
        scratch_types=[
            pltpu.VMEM((CT,), jnp.int32),
            pltpu.VMEM((CT,), jnp.int32),
            pltpu.VMEM((CT,), jnp.int32),
            pltpu.VMEM((CT,), jnp.int32),
            pltpu.VMEM((CT,), jnp.int32),
            pltpu.VMEM((CT,), jnp.int32),
            pltpu.VMEM((CT, D), jnp.float32),
            pltpu.VMEM((CT, D), jnp.float32),
            pltpu.VMEM((CT,), jnp.float32),
            pltpu.VMEM((S_CHUNK, D), jnp.float32),
            pltpu.SemaphoreType.DMA,
            pltpu.SemaphoreType.DMA,
        ],
    )
    return kfn(ids_flat, mask_flat, table)


TBLK = 128
NFULL = V // TBLK
VREM = V - NFULL * TBLK
ITERS_PER_W = NFULL // NW
EXTRA_BASE = ITERS_PER_W * NW


def _sc_detile_body(tableT_hbm, tail_hbm, out_hbm, blk0, blk1, ob0, ob1,
                    isem0, isem1, osem0, osem1):
    c = lax.axis_index("c")
    s = lax.axis_index("s")
    wid = s * 2 + c

    blks = [blk0, blk1]
    obs = [ob0, ob1]
    isems = [isem0, isem1]
    osems = [osem0, osem1]

    row16 = [lax.iota(jnp.int32, LANES) + cc * LANES for cc in range(4)]

    def fire_in(k, b):
        j = wid + NW * k
        off = pl.multiple_of(j * TBLK, TBLK)
        for i in range(D // 8):
            pltpu.async_copy(
                tableT_hbm.at[pl.ds(8 * i, 8), pl.ds(off, TBLK)],
                blks[b].at[pl.ds(8 * i, 8)], isems[b])

    def wait_in(k, b):
        j = wid + NW * k
        off = pl.multiple_of(j * TBLK, TBLK)
        for i in range(D // 8):
            pltpu.make_async_copy(
                tableT_hbm.at[pl.ds(8 * i, 8), pl.ds(off, TBLK)],
                blks[b].at[pl.ds(8 * i, 8)], isems[b]).wait()

    def fire_out(k, b):
        j = wid + NW * k
        pltpu.async_copy(obs[b], out_hbm.at[pl.ds(j * D, D)], osems[b])

    def wait_out(k, b):
        j = wid + NW * k
        pltpu.make_async_copy(
            obs[b], out_hbm.at[pl.ds(j * D, D)], osems[b]).wait()

    def transpose(b):
        blk, ob = blks[b], obs[b]

        @plsc.parallel_loop(0, D, unroll=4)
        def _(r):
            v0 = jnp.full((LANES,), 2 * r, jnp.int32)
            v1 = jnp.full((LANES,), 2 * r + 1, jnp.int32)
            for cc in range(4):
                ob[r, pl.ds(cc * LANES, LANES)] = plsc.load_gather(
                    blk, [row16[cc], v0])
                ob[r, pl.ds(D + cc * LANES, LANES)] = plsc.load_gather(
                    blk, [row16[cc], v1])

    fire_in(0, 0)

    def pair_body(kk, carry):
        for b in (0, 1):
            k = kk * 2 + b

            @pl.when(k + 1 < ITERS_PER_W)
            def _():
                fire_in(k + 1, 1 - b)

            wait_in(k, b)

            @pl.when(k >= 2)
            def _():
                wait_out(k - 2, b)

            transpose(b)
            fire_out(k, b)
        return carry

    lax.fori_loop(0, ITERS_PER_W // 2, pair_body, 0)
    wait_out(ITERS_PER_W - 2, 0)
    wait_out(ITERS_PER_W - 1, 1)

    @pl.when(wid < 4)
    def _():
        j = EXTRA_BASE + wid
        off = pl.multiple_of(j * TBLK, TBLK)
        pltpu.sync_copy(tableT_hbm.at[pl.ds(0, D), pl.ds(off, TBLK)],
                        blk0)
        transpose(0)
        pltpu.sync_copy(ob0, out_hbm.at[pl.ds(j * D, D)])

    @pl.when(wid == NW - 1)
    def _():
        pltpu.sync_copy(tail_hbm, blk1)

        def rem_row(r, carry):
            v0 = jnp.full((LANES,), 2 * r, jnp.int32)
            v1 = jnp.full((LANES,), 2 * r + 1, jnp.int32)
            for cc in range(4):
                ob1[r, pl.ds(cc * LANES, LANES)] = plsc.load_gather(
                    blk1, [row16[cc], v0])
                ob1[r, pl.ds(D + cc * LANES, LANES)] = plsc.load_gather(
                    blk1, [row16[cc], v1])
            return carry
        lax.fori_loop(0, VREM // 2, rem_row, 0)
        pltpu.sync_copy(ob1.at[pl.ds(0, VREM // 2)],
                        out_hbm.at[pl.ds(NFULL * D, VREM // 2)])


def _sc_detile(tableT, tail_pad):
    kfn = pl.kernel(
        _sc_detile_body,
        out_type=jax.ShapeDtypeStruct((V // 2, 2 * D), jnp.float32),
        mesh=plsc.VectorSubcoreMesh(core_axis_name="c", subcore_axis_name="s"),
        compiler_params=pltpu.CompilerParams(use_tc_tiling_on_sc=True,
                                             needs_layout_passes=False),
        scratch_types=[
            pltpu.VMEM((D, TBLK), jnp.float32),
            pltpu.VMEM((D, TBLK), jnp.float32),
            pltpu.VMEM((D, 2 * D), jnp.float32),
            pltpu.VMEM((D, 2 * D), jnp.float32),
            pltpu.SemaphoreType.DMA,
            pltpu.SemaphoreType.DMA,
            pltpu.SemaphoreType.DMA,
            pltpu.SemaphoreType.DMA,
        ],
    )
    return kfn(tableT, tail_pad)


VB = 16384
VB_SH = (VB // 2).bit_length() - 1
TC_GRID = -(-V // VB)
VP = TC_GRID * VB


def _tc_detile_body(xT_ref, out_ref):
    xt = xT_ref[...].T
    out_ref[...] = jnp.concatenate([xt[:VB // 2, :], xt[VB // 2:, :]], axis=1)


def _tc_detile(tableT):
    return pl.pallas_call(
        _tc_detile_body,
        grid=(TC_GRID,),
        in_specs=[pl.BlockSpec((D, VB), lambda g: (0, g))],
        out_specs=pl.BlockSpec((VB // 2, 2 * D), lambda g: (g, 0)),
        out_shape=jax.ShapeDtypeStruct((VP // 2, 2 * D), jnp.float32),
        compiler_params=pltpu.CompilerParams(
            dimension_semantics=("parallel",)),
    )(tableT)


BS = 1024


def _mlp_body(sum_ref, mask_ref, w1t_ref, b1_ref, w2p_ref, b2p_ref, out_ref):
    cnt = jnp.sum(mask_ref[...].astype(jnp.float32), axis=1, keepdims=True)
    pooled = sum_ref[...] / (cnt + 1e-9)
    h = jnp.dot(pooled, w1t_ref[...], preferred_element_type=jnp.float32)
    h = jnp.maximum(h + b1_ref[...], 0.0)
    out_ref[...] = (jnp.dot(h, w2p_ref[...], preferred_element_type=jnp.float32)
                    + b2p_ref[...])


def _mlp(pooled_sums, identity_mask, W1, b1, W2, b2):
    w1t = W1.T
    b1r = b1.reshape(1, H)
    w2p = jnp.zeros((H, 128), jnp.float32).at[:, :C].set(W2.T)
    b2p = jnp.zeros((1, 128), jnp.float32).at[0, :C].set(b2)
    out_pad = pl.pallas_call(
        _mlp_body,
        grid=(B // BS,),
        in_specs=[
            pl.BlockSpec((BS, D), lambda i: (i, 0)),
            pl.BlockSpec((BS, L), lambda i: (i, 0)),
            pl.BlockSpec((D, H), lambda i: (0, 0)),
            pl.BlockSpec((1, H), lambda i: (0, 0)),
            pl.BlockSpec((H, 128), lambda i: (0, 0)),
            pl.BlockSpec((1, 128), lambda i: (0, 0)),
        ],
        out_specs=pl.BlockSpec((BS, 128), lambda i: (i, 0)),
        out_shape=jax.ShapeDtypeStruct((B, 128), jnp.float32),
    )(pooled_sums, identity_mask, w1t, b1r, w2p, b2p)
    return out_pad[:, :C]


def kernel(input_ids, identity_mask, table, W1, b1, W2, b2):
    ids_flat = input_ids.reshape(B * L)
    mask_flat = identity_mask.reshape(B * L)
    table_lin = _tc_detile(table.T).reshape(VP, D)
    pooled_sums = _sc_pool(ids_flat, mask_flat, table_lin)
    return _mlp(pooled_sums, identity_mask, W1, b1, W2, b2)


# speedup vs baseline: 3.8918x; 1.0366x over previous
import functools

import jax
import jax.numpy as jnp
from jax import lax
from jax.experimental import pallas as pl
from jax.experimental.pallas import tpu as pltpu
from jax.experimental.pallas import tpu_sc as plsc

B, L = 16384, 50
V, D = 1000000, 64
H, C = 64, 2

NW = 32
SEQ_PER_W = B // NW
S_CHUNK = 8
CT = S_CHUNK * L
NCH = SEQ_PER_W // S_CHUNK
GATHER_SPLITS = [(0, 128), (128, 128), (256, 128), (384, 16)]
LANES = 16


def _sc_pool_body(ids_hbm, mask_hbm, table_hbm, out_hbm,
                  idx0, idx1, y0, y1, m0, m1, rows0, rows1, wbuf, outv,
                  sem0, sem1):
    c = lax.axis_index("c")
    s = lax.axis_index("s")
    wid = s * 2 + c
    tok_base = wid * (SEQ_PER_W * L)
    seq_base = wid * SEQ_PER_W

    idx = [idx0, idx1]
    ybf = [y0, y1]
    msk = [m0, m1]
    rows = [rows0, rows1]
    sems = [sem0, sem1]

    def load_and_fire(k, b):
        tb = tok_base + k * CT
        pltpu.sync_copy(ids_hbm.at[pl.ds(tb, CT)], idx[b])
        pltpu.sync_copy(mask_hbm.at[pl.ds(tb, CT)], msk[b])
        for t in range(CT // LANES):
            x16 = idx[b][pl.ds(t * LANES, LANES)]
            r16 = x16 & (VB - 1)
            ybf[b][pl.ds(t * LANES, LANES)] = (
                (x16 & ~(VB - 1))
                | ((r16 & (VB // 2 - 1)) << 1)
                | (r16 >> VB_SH))
        for (o, n) in GATHER_SPLITS:
            pltpu.async_copy(table_hbm.at[ybf[b].at[pl.ds(o, n)]],
                             rows[b].at[pl.ds(o, n)], sems[b])

    def wait_gathers(b):
        for (o, n) in GATHER_SPLITS:
            pltpu.make_async_copy(table_hbm.at[ybf[b].at[pl.ds(o, n)]],
                                  rows[b].at[pl.ds(o, n)], sems[b]).wait()

    def compute_weights(b):
        for t in range(CT // LANES):
            ids16 = idx[b][pl.ds(t * LANES, LANES)]
            mm16 = msk[b][pl.ds(t * LANES, LANES)]
            keep = (ids16 != 0) & (mm16 != 0)
            wbuf[pl.ds(t * LANES, LANES)] = jnp.where(keep, 1.0, 0.0)

    def accumulate(k, b):
        def seq_body(si, carry):
            tb = si * L
            wch = [wbuf[pl.ds(tb, LANES)],
                   wbuf[pl.ds(tb + 16, LANES)],
                   wbuf[pl.ds(tb + 32, LANES)],
                   wbuf[pl.ds(tb + 34, LANES)]]
            acc = [jnp.zeros((LANES,), jnp.float32) for _ in range(4)]
            for l in range(L):
                w = wch[l // 16][l % 16] if l < 48 else wch[3][l - 34]
                for j in range(4):
                    acc[j] = acc[j] + w * rows[b][tb + l, pl.ds(j * LANES, LANES)]
            for j in range(4):
                outv[si, pl.ds(j * LANES, LANES)] = acc[j]
            return carry
        lax.fori_loop(0, S_CHUNK, seq_body, 0)
        pltpu.sync_copy(outv, out_hbm.at[pl.ds(seq_base + k * S_CHUNK, S_CHUNK)])

    load_and_fire(0, 0)

    def pair_body(kk, carry):
        for b in (0, 1):
            k = kk * 2 + b

            @pl.when(k + 1 < NCH)
            def _():
                load_and_fire(k + 1, 1 - b)

            compute_weights(b)
            wait_gathers(b)
            accumulate(k, b)
        return carry

    lax.fori_loop(0, NCH // 2, pair_body, 0)


@functools.partial(jax.jit, static_argnames=())
def _sc_pool(ids_flat, mask_flat, table):
    kfn = pl.kernel(
        _sc_pool_body,
        out_type=jax.ShapeDtypeStruct((B, D), jnp.float32),
        mesh=plsc.VectorSubcoreMesh(core_axis_name="c", subcore_axis_name="s"),
        compiler_params=pltpu.CompilerParams(use_tc_tiling_on_sc=False),
        scratch_types=[
            pltpu.VMEM((CT,), jnp.int32),
            pltpu.VMEM((CT,), jnp.int32),
            pltpu.VMEM((CT,), jnp.int32),
            pltpu.VMEM((CT,), jnp.int32),
            pltpu.VMEM((CT,), jnp.int32),
            pltpu.VMEM((CT,), jnp.int32),
            pltpu.VMEM((CT, D), jnp.float32),
            pltpu.VMEM((CT, D), jnp.float32),
            pltpu.VMEM((CT,), jnp.float32),
            pltpu.VMEM((S_CHUNK, D), jnp.float32),
            pltpu.SemaphoreType.DMA,
            pltpu.SemaphoreType.DMA,
        ],
    )
    return kfn(ids_flat, mask_flat, table)


TBLK = 128
NFULL = V // TBLK
VREM = V - NFULL * TBLK
ITERS_PER_W = NFULL // NW
EXTRA_BASE = ITERS_PER_W * NW


def _sc_detile_body(tableT_hbm, tail_hbm, out_hbm, blk0, blk1, ob0, ob1,
                    isem0, isem1, osem0, osem1):
    c = lax.axis_index("c")
    s = lax.axis_index("s")
    wid = s * 2 + c

    blks = [blk0, blk1]
    obs = [ob0, ob1]
    isems = [isem0, isem1]
    osems = [osem0, osem1]

    row16 = [lax.iota(jnp.int32, LANES) + cc * LANES for cc in range(4)]

    def fire_in(k, b):
        j = wid + NW * k
        off = pl.multiple_of(j * TBLK, TBLK)
        for i in range(D // 8):
            pltpu.async_copy(
                tableT_hbm.at[pl.ds(8 * i, 8), pl.ds(off, TBLK)],
                blks[b].at[pl.ds(8 * i, 8)], isems[b])

    def wait_in(k, b):
        j = wid + NW * k
        off = pl.multiple_of(j * TBLK, TBLK)
        for i in range(D // 8):
            pltpu.make_async_copy(
                tableT_hbm.at[pl.ds(8 * i, 8), pl.ds(off, TBLK)],
                blks[b].at[pl.ds(8 * i, 8)], isems[b]).wait()

    def fire_out(k, b):
        j = wid + NW * k
        pltpu.async_copy(obs[b], out_hbm.at[pl.ds(j * D, D)], osems[b])

    def wait_out(k, b):
        j = wid + NW * k
        pltpu.make_async_copy(
            obs[b], out_hbm.at[pl.ds(j * D, D)], osems[b]).wait()

    def transpose(b):
        blk, ob = blks[b], obs[b]

        @plsc.parallel_loop(0, D, unroll=4)
        def _(r):
            v0 = jnp.full((LANES,), 2 * r, jnp.int32)
            v1 = jnp.full((LANES,), 2 * r + 1, jnp.int32)
            for cc in range(4):
                ob[r, pl.ds(cc * LANES, LANES)] = plsc.load_gather(
                    blk, [row16[cc], v0])
                ob[r, pl.ds(D + cc * LANES, LANES)] = plsc.load_gather(
                    blk, [row16[cc], v1])

    fire_in(0, 0)

    def pair_body(kk, carry):
        for b in (0, 1):
            k = kk * 2 + b

            @pl.when(k + 1 < ITERS_PER_W)
            def _():
                fire_in(k + 1, 1 - b)

            wait_in(k, b)

            @pl.when(k >= 2)
            def _():
                wait_out(k - 2, b)

            transpose(b)
            fire_out(k, b)
        return carry

    lax.fori_loop(0, ITERS_PER_W // 2, pair_body, 0)
    wait_out(ITERS_PER_W - 2, 0)
    wait_out(ITERS_PER_W - 1, 1)

    @pl.when(wid < 4)
    def _():
        j = EXTRA_BASE + wid
        off = pl.multiple_of(j * TBLK, TBLK)
        pltpu.sync_copy(tableT_hbm.at[pl.ds(0, D), pl.ds(off, TBLK)],
                        blk0)
        transpose(0)
        pltpu.sync_copy(ob0, out_hbm.at[pl.ds(j * D, D)])

    @pl.when(wid == NW - 1)
    def _():
        pltpu.sync_copy(tail_hbm, blk1)

        def rem_row(r, carry):
            v0 = jnp.full((LANES,), 2 * r, jnp.int32)
            v1 = jnp.full((LANES,), 2 * r + 1, jnp.int32)
            for cc in range(4):
                ob1[r, pl.ds(cc * LANES, LANES)] = plsc.load_gather(
                    blk1, [row16[cc], v0])
                ob1[r, pl.ds(D + cc * LANES, LANES)] = plsc.load_gather(
                    blk1, [row16[cc], v1])
            return carry
        lax.fori_loop(0, VREM // 2, rem_row, 0)
        pltpu.sync_copy(ob1.at[pl.ds(0, VREM // 2)],
                        out_hbm.at[pl.ds(NFULL * D, VREM // 2)])


def _sc_detile(tableT, tail_pad):
    kfn = pl.kernel(
        _sc_detile_body,
        out_type=jax.ShapeDtypeStruct((V // 2, 2 * D), jnp.float32),
        mesh=plsc.VectorSubcoreMesh(core_axis_name="c", subcore_axis_name="s"),
        compiler_params=pltpu.CompilerParams(use_tc_tiling_on_sc=True,
                                             needs_layout_passes=False),
        scratch_types=[
            pltpu.VMEM((D, TBLK), jnp.float32),
            pltpu.VMEM((D, TBLK), jnp.float32),
            pltpu.VMEM((D, 2 * D), jnp.float32),
            pltpu.VMEM((D, 2 * D), jnp.float32),
            pltpu.SemaphoreType.DMA,
            pltpu.SemaphoreType.DMA,
            pltpu.SemaphoreType.DMA,
            pltpu.SemaphoreType.DMA,
        ],
    )
    return kfn(tableT, tail_pad)


VB = 32768
VB_SH = (VB // 2).bit_length() - 1
TC_GRID = -(-V // VB)
VP = TC_GRID * VB


def _tc_detile_body(xT_ref, out_ref):
    xt = xT_ref[...].T
    out_ref[...] = jnp.concatenate([xt[:VB // 2, :], xt[VB // 2:, :]], axis=1)


def _tc_detile(tableT):
    return pl.pallas_call(
        _tc_detile_body,
        grid=(TC_GRID,),
        in_specs=[pl.BlockSpec((D, VB), lambda g: (0, g))],
        out_specs=pl.BlockSpec((VB // 2, 2 * D), lambda g: (g, 0)),
        out_shape=jax.ShapeDtypeStruct((VP // 2, 2 * D), jnp.float32),
        compiler_params=pltpu.CompilerParams(
            dimension_semantics=("parallel",)),
    )(tableT)


BS = 1024


def _mlp_body(sum_ref, mask_ref, w1t_ref, b1_ref, w2p_ref, b2p_ref, out_ref):
    cnt = jnp.sum(mask_ref[...].astype(jnp.float32), axis=1, keepdims=True)
    pooled = sum_ref[...] / (cnt + 1e-9)
    h = jnp.dot(pooled, w1t_ref[...], preferred_element_type=jnp.float32)
    h = jnp.maximum(h + b1_ref[...], 0.0)
    out_ref[...] = (jnp.dot(h, w2p_ref[...], preferred_element_type=jnp.float32)
                    + b2p_ref[...])


def _mlp(pooled_sums, identity_mask, W1, b1, W2, b2):
    w1t = W1.T
    b1r = b1.reshape(1, H)
    w2p = jnp.zeros((H, 128), jnp.float32).at[:, :C].set(W2.T)
    b2p = jnp.zeros((1, 128), jnp.float32).at[0, :C].set(b2)
    out_pad = pl.pallas_call(
        _mlp_body,
        grid=(B // BS,),
        in_specs=[
            pl.BlockSpec((BS, D), lambda i: (i, 0)),
            pl.BlockSpec((BS, L), lambda i: (i, 0)),
            pl.BlockSpec((D, H), lambda i: (0, 0)),
            pl.BlockSpec((1, H), lambda i: (0, 0)),
            pl.BlockSpec((H, 128), lambda i: (0, 0)),
            pl.BlockSpec((1, 128), lambda i: (0, 0)),
        ],
        out_specs=pl.BlockSpec((BS, 128), lambda i: (i, 0)),
        out_shape=jax.ShapeDtypeStruct((B, 128), jnp.float32),
    )(pooled_sums, identity_mask, w1t, b1r, w2p, b2p)
    return out_pad[:, :C]


def kernel(input_ids, identity_mask, table, W1, b1, W2, b2):
    ids_flat = input_ids.reshape(B * L)
    mask_flat = identity_mask.reshape(B * L)
    table_lin = _tc_detile(table.T).reshape(VP, D)
    pooled_sums = _sc_pool(ids_flat, mask_flat, table_lin)
    return _mlp(pooled_sums, identity_mask, W1, b1, W2, b2)


# speedup vs baseline: 3.9197x; 1.0072x over previous
import functools

import jax
import jax.numpy as jnp
from jax import lax
from jax.experimental import pallas as pl
from jax.experimental.pallas import tpu as pltpu
from jax.experimental.pallas import tpu_sc as plsc

B, L = 16384, 50
V, D = 1000000, 64
H, C = 64, 2

NW = 32
SEQ_PER_W = B // NW
S_CHUNK = 8
CT = S_CHUNK * L
NCH = SEQ_PER_W // S_CHUNK
GATHER_SPLITS = [(0, 128), (128, 128), (256, 128), (384, 16)]
LANES = 16


def _sc_pool_body(ids_hbm, mask_hbm, table_hbm, out_hbm,
                  idx0, idx1, y0, y1, m0, m1, rows0, rows1, wbuf, outv,
                  sem0, sem1):
    c = lax.axis_index("c")
    s = lax.axis_index("s")
    wid = s * 2 + c
    tok_base = wid * (SEQ_PER_W * L)
    seq_base = wid * SEQ_PER_W

    idx = [idx0, idx1]
    ybf = [y0, y1]
    msk = [m0, m1]
    rows = [rows0, rows1]
    sems = [sem0, sem1]

    def load_and_fire(k, b):
        tb = tok_base + k * CT
        pltpu.sync_copy(ids_hbm.at[pl.ds(tb, CT)], idx[b])
        pltpu.sync_copy(mask_hbm.at[pl.ds(tb, CT)], msk[b])
        for t in range(CT // LANES):
            x16 = idx[b][pl.ds(t * LANES, LANES)]
            r16 = x16 & (VB - 1)
            ybf[b][pl.ds(t * LANES, LANES)] = (
                (x16 & ~(VB - 1))
                | ((r16 & (VB // 2 - 1)) << 1)
                | (r16 >> VB_SH))
        for (o, n) in GATHER_SPLITS:
            pltpu.async_copy(table_hbm.at[ybf[b].at[pl.ds(o, n)]],
                             rows[b].at[pl.ds(o, n)], sems[b])

    def wait_gathers(b):
        for (o, n) in GATHER_SPLITS:
            pltpu.make_async_copy(table_hbm.at[ybf[b].at[pl.ds(o, n)]],
                                  rows[b].at[pl.ds(o, n)], sems[b]).wait()

    def compute_weights(b):
        for t in range(CT // LANES):
            ids16 = idx[b][pl.ds(t * LANES, LANES)]
            mm16 = msk[b][pl.ds(t * LANES, LANES)]
            keep = (ids16 != 0) & (mm16 != 0)
            wbuf[pl.ds(t * LANES, LANES)] = jnp.where(keep, 1.0, 0.0)

    def accumulate(k, b):
        def seq_body(si, carry):
            tb = si * L
            wch = [wbuf[pl.ds(tb, LANES)],
                   wbuf[pl.ds(tb + 16, LANES)],
                   wbuf[pl.ds(tb + 32, LANES)],
                   wbuf[pl.ds(tb + 34, LANES)]]
            acc = [jnp.zeros((LANES,), jnp.float32) for _ in range(4)]
            for l in range(L):
                w = wch[l // 16][l % 16] if l < 48 else wch[3][l - 34]
                for j in range(4):
                    acc[j] = acc[j] + w * rows[b][tb + l, pl.ds(j * LANES, LANES)]
            for j in range(4):
                outv[si, pl.ds(j * LANES, LANES)] = acc[j]
            return carry
        lax.fori_loop(0, S_CHUNK, seq_body, 0)
        pltpu.sync_copy(outv, out_hbm.at[pl.ds(seq_base + k * S_CHUNK, S_CHUNK)])

    load_and_fire(0, 0)

    def pair_body(kk, carry):
        for b in (0, 1):
            k = kk * 2 + b

            @pl.when(k + 1 < NCH)
            def _():
                load_and_fire(k + 1, 1 - b)

            compute_weights(b)
            wait_gathers(b)
            accumulate(k, b)
        return carry

    lax.fori_loop(0, NCH // 2, pair_body, 0)


@functools.partial(jax.jit, static_argnames=())
def _sc_pool(ids_flat, mask_flat, table):
    kfn = pl.kernel(
        _sc_pool_body,
        out_type=jax.ShapeDtypeStruct((B, D), jnp.float32),
        mesh=plsc.VectorSubcoreMesh(core_axis_name="c", subcore_axis_name="s"),
        compiler_params=pltpu.CompilerParams(use_tc_tiling_on_sc=False),
        scratch_types=[
            pltpu.VMEM((CT,), jnp.int32),
            pltpu.VMEM((CT,), jnp.int32),
            pltpu.VMEM((CT,), jnp.int32),
            pltpu.VMEM((CT,), jnp.int32),
            pltpu.VMEM((CT,), jnp.int32),
            pltpu.VMEM((CT,), jnp.int32),
            pltpu.VMEM((CT, D), jnp.float32),
            pltpu.VMEM((CT, D), jnp.float32),
            pltpu.VMEM((CT,), jnp.float32),
            pltpu.VMEM((S_CHUNK, D), jnp.float32),
            pltpu.SemaphoreType.DMA,
            pltpu.SemaphoreType.DMA,
        ],
    )
    return kfn(ids_flat, mask_flat, table)


VB = 32768
VB_SH = (VB // 2).bit_length() - 1
TC_GRID = -(-V // VB)
VP = TC_GRID * VB


def _tc_detile_body(xT_ref, out_ref):
    xt = xT_ref[...].T
    out_ref[...] = jnp.concatenate([xt[:VB // 2, :], xt[VB // 2:, :]], axis=1)


def _tc_detile(tableT):
    return pl.pallas_call(
        _tc_detile_body,
        grid=(TC_GRID,),
        in_specs=[pl.BlockSpec((D, VB), lambda g: (0, g))],
        out_specs=pl.BlockSpec((VB // 2, 2 * D), lambda g: (g, 0)),
        out_shape=jax.ShapeDtypeStruct((VP // 2, 2 * D), jnp.float32),
        compiler_params=pltpu.CompilerParams(
            dimension_semantics=("parallel",)),
    )(tableT)


BS = 1024


def _mlp_body(sum_ref, mask_ref, w1t_ref, b1_ref, w2p_ref, b2p_ref, out_ref):
    cnt = jnp.sum(mask_ref[...].astype(jnp.float32), axis=1, keepdims=True)
    pooled = sum_ref[...] / (cnt + 1e-9)
    h = jnp.dot(pooled, w1t_ref[...], preferred_element_type=jnp.float32)
    h = jnp.maximum(h + b1_ref[...], 0.0)
    out_ref[...] = (jnp.dot(h, w2p_ref[...], preferred_element_type=jnp.float32)
                    + b2p_ref[...])


def _mlp(pooled_sums, identity_mask, W1, b1, W2, b2):
    w1t = W1.T
    b1r = b1.reshape(1, H)
    w2p = jnp.zeros((H, 128), jnp.float32).at[:, :C].set(W2.T)
    b2p = jnp.zeros((1, 128), jnp.float32).at[0, :C].set(b2)
    out_pad = pl.pallas_call(
        _mlp_body,
        grid=(B // BS,),
        in_specs=[
            pl.BlockSpec((BS, D), lambda i: (i, 0)),
            pl.BlockSpec((BS, L), lambda i: (i, 0)),
            pl.BlockSpec((D, H), lambda i: (0, 0)),
            pl.BlockSpec((1, H), lambda i: (0, 0)),
            pl.BlockSpec((H, 128), lambda i: (0, 0)),
            pl.BlockSpec((1, 128), lambda i: (0, 0)),
        ],
        out_specs=pl.BlockSpec((BS, 128), lambda i: (i, 0)),
        out_shape=jax.ShapeDtypeStruct((B, 128), jnp.float32),
    )(pooled_sums, identity_mask, w1t, b1r, w2p, b2p)
    return out_pad[:, :C]


def kernel(input_ids, identity_mask, table, W1, b1, W2, b2):
    ids_flat = input_ids.reshape(B * L)
    mask_flat = identity_mask.reshape(B * L)
    table_lin = _tc_detile(table.T).reshape(VP, D)
    pooled_sums = _sc_pool(ids_flat, mask_flat, table_lin)
    return _mlp(pooled_sums, identity_mask, W1, b1, W2, b2)
